# baseline (speedup 1.0000x reference)
import functools

import jax
import jax.numpy as jnp
from jax import lax
from jax.experimental import pallas as pl
from jax.experimental.pallas import tpu as pltpu
from jax.experimental.pallas import tpu_sc as plsc


def kernel(inputs, pos_embedding):
    B, S = inputs.shape
    D = pos_embedding.shape[1]

    mesh = plsc.ScalarSubcoreMesh(axis_name="c", num_cores=2)
    rows_core = S // 2
    Rc = 512
    n_chunks = rows_core // Rc

    @functools.partial(
        pl.kernel,
        mesh=mesh,
        out_type=jax.ShapeDtypeStruct((B * S, D), jnp.float32),
        scratch_types=[
            pltpu.VMEM_SHARED((Rc, D), jnp.float32),
            pltpu.VMEM_SHARED((Rc, D), jnp.float32),
            pltpu.SemaphoreType.DMA,
            pltpu.SemaphoreType.DMA,
            pltpu.SemaphoreType.DMA,
        ],
    )
    def sc_broadcast(table_hbm, out_hbm, buf0, buf1, rsem, wsem0, wsem1):
        cid = lax.axis_index("c")
        base = cid * rows_core
        bufs = (buf0, buf1)
        wsems = (wsem0, wsem1)
        reads = [None] * n_chunks
        writes = [None] * n_chunks
        reads[0] = pltpu.async_copy(table_hbm.at[pl.ds(base, Rc)], bufs[0],
                                    rsem)
        for c in range(n_chunks):
            off = base + c * Rc
            reads[c].wait()
            if c >= 1:
                for w in writes[c - 1]:
                    w.wait()
            if c + 1 < n_chunks:
                reads[c + 1] = pltpu.async_copy(
                    table_hbm.at[pl.ds(off + Rc, Rc)], bufs[(c + 1) % 2],
                    rsem)
            writes[c] = [
                pltpu.async_copy(
                    bufs[c % 2], out_hbm.at[pl.ds(b * S + off, Rc)],
                    wsems[c % 2])
                for b in range(B)
            ]
        for w in writes[n_chunks - 1]:
            w.wait()

    return sc_broadcast(pos_embedding).reshape(B, S, D)


# speedup vs baseline: 1.3554x; 1.3554x over previous
import functools

import jax
import jax.numpy as jnp
from jax import lax
from jax.experimental import pallas as pl
from jax.experimental.pallas import tpu as pltpu
from jax.experimental.pallas import tpu_sc as plsc


def kernel(inputs, pos_embedding):
    B, S = inputs.shape
    D = pos_embedding.shape[1]

    mesh = plsc.VectorSubcoreMesh(core_axis_name="c", subcore_axis_name="s")
    NC, NS = mesh.num_cores, mesh.num_subcores
    NW = NC * NS
    rows_w = S // NW
    R = min(rows_w, 64)
    n_chunks = rows_w // R

    @functools.partial(
        pl.kernel,
        mesh=mesh,
        out_type=jax.ShapeDtypeStruct((B * S, D), jnp.float32),
        scratch_types=[
            pltpu.VMEM((R, D), jnp.float32),
            pltpu.SemaphoreType.DMA,
        ],
    )
    def sc_broadcast(table_hbm, out_hbm, buf, sem):
        wid = lax.axis_index("s") * NC + lax.axis_index("c")
        base = wid * rows_w
        for c in range(n_chunks):
            off = base + c * R
            pltpu.async_copy(table_hbm.at[pl.ds(off, R)], buf, sem).wait()
            for b in range(B):
                pltpu.sync_copy(buf, out_hbm.at[pl.ds(b * S + off, R)])

    return sc_broadcast(pos_embedding).reshape(B, S, D)
